# baseline (speedup 1.0000x reference)
import functools

import jax
import jax.numpy as jnp
from jax import lax
from jax.experimental import pallas as pl
from jax.experimental.pallas import tpu as pltpu
from jax.experimental.pallas import tpu_sc as plsc

N = 10000
E = 320000
D = 128
DO = 64
DH = D // 2

NC = 2
NS = 16
NW = NC * NS

NPAD = 10240
RPT = NPAD // NS
ZB = 80
EB = 125
DBLK = (E // NW) // EB
PBLK = (E // NS) // EB
NBUF = 4
DEGW = 16


def _mesh():
    return plsc.VectorSubcoreMesh(
        core_axis_name="c", subcore_axis_name="s", num_cores=NC, num_subcores=NS)


def _sc_degree_body(dst_hbm, out_hbm, idxs_v, ones_v, acc_sh, sem):
    cid = lax.axis_index("c")
    sid = lax.axis_index("s")
    wid = sid * NC + cid

    zero16 = jnp.zeros((16,), jnp.float32)
    for i in range(EB):
        ones_v[i, :] = zero16
    for k in range(RPT // ZB):
        pltpu.sync_copy(ones_v.at[pl.ds(0, ZB)],
                        acc_sh.at[pl.ds(sid * RPT + k * ZB, ZB)])
    one16 = jnp.ones((16,), jnp.float32)
    for i in range(EB):
        ones_v[i, :] = one16
    pltpu.sync_copy(dst_hbm.at[wid], idxs_v)
    plsc.subcore_barrier()

    FIRE = 8

    def body(r, carry):
        for k in range(FIRE):
            pltpu.async_copy(ones_v, acc_sh.at[idxs_v.at[r * FIRE + k]], sem,
                             add=True)
        for k in range(FIRE):
            pltpu.make_async_copy(ones_v, acc_sh.at[idxs_v.at[r * FIRE + k]],
                                  sem).wait()
        return carry

    lax.fori_loop(0, DBLK // FIRE, body, 0)
    plsc.subcore_barrier()
    pltpu.sync_copy(acc_sh.at[pl.ds(sid * RPT, RPT)],
                    out_hbm.at[cid, pl.ds(sid * RPT, RPT)])


@functools.cache
def _sc_degree():
    return pl.kernel(
        _sc_degree_body,
        mesh=_mesh(),
        compiler_params=pltpu.CompilerParams(use_tc_tiling_on_sc=False),
        out_type=jax.ShapeDtypeStruct((NC, NPAD, DEGW), jnp.float32),
        scratch_types=[
            pltpu.VMEM((DBLK, EB), jnp.int32),
            pltpu.VMEM((EB, DEGW), jnp.float32),
            pltpu.VMEM_SHARED((NPAD, DEGW), jnp.float32),
            pltpu.SemaphoreType.DMA,
        ],
    )


def _sc_prop_body(f0_hbm, f1_hbm, src_hbm, dst_hbm, out_hbm, srcs_v, dsts_v,
                  rows_v, acc_sh, gsems, ssems):
    cid = lax.axis_index("c")
    sid = lax.axis_index("s")

    zero16 = jnp.zeros((16,), jnp.float32)
    for i in range(EB):
        for j in range(DH // 16):
            rows_v[0][i, pl.ds(j * 16, 16)] = zero16
    for k in range(RPT // ZB):
        pltpu.sync_copy(rows_v[0].at[pl.ds(0, ZB)],
                        acc_sh.at[pl.ds(sid * RPT + k * ZB, ZB)])
    pltpu.sync_copy(src_hbm.at[sid], srcs_v)
    pltpu.sync_copy(dst_hbm.at[sid], dsts_v)
    plsc.subcore_barrier()

    def edge_loop(f_hbm):
        for b in range(NBUF):
            pltpu.async_copy(f_hbm.at[srcs_v.at[b]], rows_v[b], gsems[b])

        def rnd(r, carry):
            for b in range(NBUF):
                j = NBUF * r + b
                pltpu.make_async_copy(f_hbm.at[srcs_v.at[j]], rows_v[b],
                                      gsems[b]).wait()
                pltpu.async_copy(rows_v[b], acc_sh.at[dsts_v.at[j]], ssems[b],
                                 add=True)
                b2 = (b + 2) % NBUF

                @pl.when(jnp.logical_and(j >= 2, j + 2 < PBLK))
                def _():
                    pltpu.make_async_copy(rows_v[b2], acc_sh.at[dsts_v.at[j]],
                                          ssems[b2]).wait()
                    pltpu.async_copy(f_hbm.at[srcs_v.at[j + 2]], rows_v[b2],
                                     gsems[b2])
            return carry

        lax.fori_loop(0, PBLK // NBUF, rnd, 0)
        for b in range(NBUF):
            pltpu.make_async_copy(rows_v[b], acc_sh.at[dsts_v.at[0]],
                                  ssems[b]).wait()

    @pl.when(cid == 0)
    def _():
        edge_loop(f0_hbm)

    @pl.when(cid == 1)
    def _():
        edge_loop(f1_hbm)

    plsc.subcore_barrier()
    pltpu.sync_copy(acc_sh.at[pl.ds(sid * RPT, RPT)],
                    out_hbm.at[cid, pl.ds(sid * RPT, RPT)])


@functools.cache
def _sc_prop():
    return pl.kernel(
        _sc_prop_body,
        mesh=_mesh(),
        compiler_params=pltpu.CompilerParams(use_tc_tiling_on_sc=False),
        out_type=jax.ShapeDtypeStruct((NC, NPAD, DH), jnp.float32),
        scratch_types=[
            pltpu.VMEM((PBLK, EB), jnp.int32),
            pltpu.VMEM((PBLK, EB), jnp.int32),
            [pltpu.VMEM((EB, DH), jnp.float32)] * NBUF,
            pltpu.VMEM_SHARED((NPAD, DH), jnp.float32),
            [pltpu.SemaphoreType.DMA] * NBUF,
            [pltpu.SemaphoreType.DMA] * NBUF,
        ],
    )


R = 512
NG = (N + R - 1) // R


def _dinv_block(degp_ref):
    d = degp_ref[0, :, 0:1] + degp_ref[1, :, 0:1] + 1.0
    return lax.rsqrt(d)


def _t1_body(degp_ref, x_ref, xps_ref):
    xp = _dinv_block(degp_ref) * x_ref[...]
    xps_ref[0] = xp[:, :DH]
    xps_ref[1] = xp[:, DH:]


_t1 = pl.pallas_call(
    _t1_body,
    grid=(NG,),
    in_specs=[
        pl.BlockSpec((NC, R, DEGW), lambda i: (0, i, 0)),
        pl.BlockSpec((R, D), lambda i: (i, 0)),
    ],
    out_specs=pl.BlockSpec((NC, R, DH), lambda i: (0, i, 0)),
    out_shape=jax.ShapeDtypeStruct((NC, N, DH), jnp.float32),
)


def _t2_body(acc_ref, xps_ref, degp_ref, w_ref, b_ref, hps_ref):
    dinv = _dinv_block(degp_ref)
    a = jnp.concatenate([acc_ref[0] + xps_ref[0], acc_ref[1] + xps_ref[1]],
                        axis=1)
    s = dinv * a
    h = jnp.dot(s, w_ref[...], preferred_element_type=jnp.float32) + b_ref[...]
    hp = dinv * jnp.maximum(h, 0.0)
    hps_ref[0] = hp[:, :DH]
    hps_ref[1] = hp[:, DH:]


_t2 = pl.pallas_call(
    _t2_body,
    grid=(NG,),
    in_specs=[
        pl.BlockSpec((NC, R, DH), lambda i: (0, i, 0)),
        pl.BlockSpec((NC, R, DH), lambda i: (0, i, 0)),
        pl.BlockSpec((NC, R, DEGW), lambda i: (0, i, 0)),
        pl.BlockSpec((D, D), lambda i: (0, 0)),
        pl.BlockSpec((1, D), lambda i: (0, 0)),
    ],
    out_specs=pl.BlockSpec((NC, R, DH), lambda i: (0, i, 0)),
    out_shape=jax.ShapeDtypeStruct((NC, N, DH), jnp.float32),
)


def _t3_body(acc_ref, hps_ref, degp_ref, wmu_ref, bmu_ref, wls_ref, bls_ref,
             mu_ref, ls_ref):
    dinv = _dinv_block(degp_ref)
    a = jnp.concatenate([acc_ref[0] + hps_ref[0], acc_ref[1] + hps_ref[1]],
                        axis=1)
    g = dinv * a
    mu_ref[...] = jnp.dot(g, wmu_ref[...], preferred_element_type=jnp.float32) + bmu_ref[...]
    ls_ref[...] = jnp.dot(g, wls_ref[...], preferred_element_type=jnp.float32) + bls_ref[...]


_t3 = pl.pallas_call(
    _t3_body,
    grid=(NG,),
    in_specs=[
        pl.BlockSpec((NC, R, DH), lambda i: (0, i, 0)),
        pl.BlockSpec((NC, R, DH), lambda i: (0, i, 0)),
        pl.BlockSpec((NC, R, DEGW), lambda i: (0, i, 0)),
        pl.BlockSpec((D, DO), lambda i: (0, 0)),
        pl.BlockSpec((1, DO), lambda i: (0, 0)),
        pl.BlockSpec((D, DO), lambda i: (0, 0)),
        pl.BlockSpec((1, DO), lambda i: (0, 0)),
    ],
    out_specs=[
        pl.BlockSpec((R, DO), lambda i: (i, 0)),
        pl.BlockSpec((R, DO), lambda i: (i, 0)),
    ],
    out_shape=[
        jax.ShapeDtypeStruct((N, DO), jnp.float32),
        jax.ShapeDtypeStruct((N, DO), jnp.float32),
    ],
)


def kernel(x, edge_index, W1, b1, Wmu, bmu, Wls, bls):
    src_p = edge_index[0].reshape(NS, PBLK, EB)
    dst_p = edge_index[1].reshape(NS, PBLK, EB)
    dst_d = edge_index[1].reshape(NW, DBLK, EB)
    degp = _sc_degree()(dst_d)
    xps = _t1(degp, x)
    prop = _sc_prop()
    acc1 = prop(xps[0], xps[1], src_p, dst_p)
    hps = _t2(acc1, xps, degp, W1, b1.reshape(1, D))
    acc2 = prop(hps[0], hps[1], src_p, dst_p)
    mu, logstd = _t3(acc2, hps, degp, Wmu, bmu.reshape(1, DO),
                     Wls, bls.reshape(1, DO))
    return (mu, logstd)


# speedup vs baseline: 1.1529x; 1.1529x over previous
import functools

import jax
import jax.numpy as jnp
from jax import lax
from jax.experimental import pallas as pl
from jax.experimental.pallas import tpu as pltpu
from jax.experimental.pallas import tpu_sc as plsc

N = 10000
E = 320000
D = 128
DO = 64

NC = 2
NS = 16
NW = NC * NS

NPAD = 10240
ROWS_PER_TILE = NPAD // NS
EB = 125
EPW = E // NW
NBLK = EPW // EB
ZB = 80
DEGW = 16

EBP = 80
PBLK = EPW // EBP
NBUF = 4
IS = 8

def _mesh():
    return plsc.VectorSubcoreMesh(
        core_axis_name="c", subcore_axis_name="s", num_cores=NC, num_subcores=NS)


def _sc_degree_body(dst_hbm, out_hbm, idxs_v, ones_v, acc_sh, sem):
    cid = lax.axis_index("c")
    sid = lax.axis_index("s")
    wid = sid * NC + cid

    zero16 = jnp.zeros((16,), jnp.float32)
    for i in range(EB):
        ones_v[i, :] = zero16
    zslice = ones_v.at[pl.ds(0, ZB)]
    for k in range(ROWS_PER_TILE // ZB):
        pltpu.sync_copy(zslice, acc_sh.at[pl.ds(sid * ROWS_PER_TILE + k * ZB, ZB)])
    one16 = jnp.ones((16,), jnp.float32)
    for i in range(EB):
        ones_v[i, :] = one16
    pltpu.sync_copy(dst_hbm.at[wid], idxs_v)
    plsc.subcore_barrier()

    FIRE = 8

    def body(r, carry):
        for k in range(FIRE):
            pltpu.async_copy(ones_v, acc_sh.at[idxs_v.at[r * FIRE + k]], sem,
                             add=True)
        for k in range(FIRE):
            pltpu.make_async_copy(ones_v, acc_sh.at[idxs_v.at[r * FIRE + k]],
                                  sem).wait()
        return carry

    lax.fori_loop(0, NBLK // FIRE, body, 0)
    plsc.subcore_barrier()
    pltpu.sync_copy(acc_sh.at[pl.ds(sid * ROWS_PER_TILE, ROWS_PER_TILE)],
                    out_hbm.at[cid, pl.ds(sid * ROWS_PER_TILE, ROWS_PER_TILE)])


@functools.cache
def _sc_degree():
    return pl.kernel(
        _sc_degree_body,
        mesh=_mesh(),
        out_type=jax.ShapeDtypeStruct((NC, NPAD, DEGW), jnp.float32),
        scratch_types=[
            pltpu.VMEM((NBLK, EB), jnp.int32),
            pltpu.VMEM((EB, DEGW), jnp.float32),
            pltpu.VMEM_SHARED((NPAD, DEGW), jnp.float32),
            pltpu.SemaphoreType.DMA,
        ],
    )


def _sc_prop_body(f_hbm, src_hbm, dst_hbm, out_hbm, srcs_v, dsts_v,
                  rows_v, acc_sh, isems, gsems, ssems):
    cid = lax.axis_index("c")
    sid = lax.axis_index("s")
    wid = sid * NC + cid

    zero16 = jnp.zeros((16,), jnp.float32)
    for i in range(ZB):
        for j in range(D // 16):
            rows_v[0][i, pl.ds(j * 16, 16)] = zero16
    for k in range(ROWS_PER_TILE // ZB):
        pltpu.sync_copy(rows_v[0], acc_sh.at[pl.ds(sid * ROWS_PER_TILE + k * ZB, ZB)])

    def idx_load(t, j):
        pltpu.async_copy(src_hbm.at[wid, pl.ds(j, 1)], srcs_v.at[pl.ds(t, 1)],
                         isems[t])
        pltpu.async_copy(dst_hbm.at[wid, pl.ds(j, 1)], dsts_v.at[pl.ds(t, 1)],
                         isems[t])

    def idx_wait(t, j):
        pltpu.make_async_copy(src_hbm.at[wid, pl.ds(j, 1)],
                              srcs_v.at[pl.ds(t, 1)], isems[t]).wait()
        pltpu.make_async_copy(dst_hbm.at[wid, pl.ds(j, 1)],
                              dsts_v.at[pl.ds(t, 1)], isems[t]).wait()

    def gather(b, t):
        pltpu.async_copy(f_hbm.at[srcs_v.at[t]], rows_v[b], gsems[b])

    def gather_wait(b, t):
        pltpu.make_async_copy(f_hbm.at[srcs_v.at[t]], rows_v[b], gsems[b]).wait()

    def scatter(b, t):
        pltpu.async_copy(rows_v[b], acc_sh.at[dsts_v.at[t]], ssems[b], add=True)

    def scatter_wait(b):
        pltpu.make_async_copy(rows_v[b], acc_sh.at[dsts_v.at[0]], ssems[b]).wait()

    for t in range(IS):
        idx_load(t, t)
    plsc.subcore_barrier()
    for b in range(NBUF):
        idx_wait(b, b)
        gather(b, b)

    ROUNDS = (PBLK - (PBLK % IS)) // IS
    TAIL = PBLK % IS

    def rnd(r, carry):
        for k in range(IS):
            j = IS * r + k
            b = k % NBUF
            b2 = (b + 2) % NBUF
            s2 = (k + 2) % IS
            gather_wait(b, k)
            scatter(b, k)

            def prefetch():
                scatter_wait(b2)
                idx_wait(s2, j + 2)
                gather(b2, s2)

            if k < 2:
                pl.when(r > 0)(prefetch)
            else:
                prefetch()

            def iload():
                idx_load((k + 6) % IS, j + 6)

            if k < 2:
                pl.when(r > 0)(iload)
            elif IS * (ROUNDS - 1) + k + 6 >= PBLK:
                pl.when(r < ROUNDS - 1)(iload)
            else:
                iload()
        return carry

    lax.fori_loop(0, ROUNDS, rnd, 0)

    j0 = ROUNDS * IS
    for k in range(TAIL):
        j = j0 + k
        b = k % NBUF
        gather_wait(b, k)
        scatter(b, k)
        if k + 2 < TAIL:
            b2 = (b + 2) % NBUF
            s2 = (k + 2) % IS
            scatter_wait(b2)
            idx_wait(s2, j + 2)
            gather(b2, s2)
    for k in range(NBUF):
        scatter_wait((TAIL + k) % NBUF)

    plsc.subcore_barrier()
    pltpu.sync_copy(acc_sh.at[pl.ds(sid * ROWS_PER_TILE, ROWS_PER_TILE)],
                    out_hbm.at[cid, pl.ds(sid * ROWS_PER_TILE, ROWS_PER_TILE)])


@functools.cache
def _sc_prop():
    return pl.kernel(
        _sc_prop_body,
        mesh=_mesh(),
        out_type=jax.ShapeDtypeStruct((NC, NPAD, D), jnp.float32),
        scratch_types=[
            pltpu.VMEM((IS, EBP), jnp.int32),
            pltpu.VMEM((IS, EBP), jnp.int32),
            [pltpu.VMEM((EBP, D), jnp.float32)] * NBUF,
            pltpu.VMEM_SHARED((NPAD, D), jnp.float32),
            [pltpu.SemaphoreType.DMA] * IS,
            [pltpu.SemaphoreType.DMA] * NBUF,
            [pltpu.SemaphoreType.DMA] * NBUF,
        ],
    )


R = 512
GRID = (NPAD // R,)


def _dinv_block(degp_ref):
    d = degp_ref[0, :, 0:1] + degp_ref[1, :, 0:1] + 1.0
    return lax.rsqrt(d)


def _t1_body(degp_ref, x_ref, xp_ref):
    xp_ref[...] = _dinv_block(degp_ref) * x_ref[...]


_t1 = pl.pallas_call(
    _t1_body,
    grid=GRID,
    in_specs=[
        pl.BlockSpec((NC, R, DEGW), lambda i: (0, i, 0)),
        pl.BlockSpec((R, D), lambda i: (i, 0)),
    ],
    out_specs=pl.BlockSpec((R, D), lambda i: (i, 0)),
    out_shape=jax.ShapeDtypeStruct((N, D), jnp.float32),
)


def _t2_body(acc_ref, xp_ref, degp_ref, w_ref, b_ref, hp_ref):
    dinv = _dinv_block(degp_ref)
    s = dinv * (acc_ref[0] + acc_ref[1] + xp_ref[...])
    h = jnp.dot(s, w_ref[...], preferred_element_type=jnp.float32) + b_ref[...]
    hp_ref[...] = dinv * jnp.maximum(h, 0.0)


_t2 = pl.pallas_call(
    _t2_body,
    grid=GRID,
    in_specs=[
        pl.BlockSpec((NC, R, D), lambda i: (0, i, 0)),
        pl.BlockSpec((R, D), lambda i: (i, 0)),
        pl.BlockSpec((NC, R, DEGW), lambda i: (0, i, 0)),
        pl.BlockSpec((D, D), lambda i: (0, 0)),
        pl.BlockSpec((1, D), lambda i: (0, 0)),
    ],
    out_specs=pl.BlockSpec((R, D), lambda i: (i, 0)),
    out_shape=jax.ShapeDtypeStruct((N, D), jnp.float32),
)


def _t3_body(acc_ref, hp_ref, degp_ref, wmu_ref, bmu_ref, wls_ref, bls_ref,
             mu_ref, ls_ref):
    dinv = _dinv_block(degp_ref)
    g = dinv * (acc_ref[0] + acc_ref[1] + hp_ref[...])
    mu_ref[...] = jnp.dot(g, wmu_ref[...], preferred_element_type=jnp.float32) + bmu_ref[...]
    ls_ref[...] = jnp.dot(g, wls_ref[...], preferred_element_type=jnp.float32) + bls_ref[...]


_t3 = pl.pallas_call(
    _t3_body,
    grid=GRID,
    in_specs=[
        pl.BlockSpec((NC, R, D), lambda i: (0, i, 0)),
        pl.BlockSpec((R, D), lambda i: (i, 0)),
        pl.BlockSpec((NC, R, DEGW), lambda i: (0, i, 0)),
        pl.BlockSpec((D, DO), lambda i: (0, 0)),
        pl.BlockSpec((1, DO), lambda i: (0, 0)),
        pl.BlockSpec((D, DO), lambda i: (0, 0)),
        pl.BlockSpec((1, DO), lambda i: (0, 0)),
    ],
    out_specs=[
        pl.BlockSpec((R, DO), lambda i: (i, 0)),
        pl.BlockSpec((R, DO), lambda i: (i, 0)),
    ],
    out_shape=[
        jax.ShapeDtypeStruct((N, DO), jnp.float32),
        jax.ShapeDtypeStruct((N, DO), jnp.float32),
    ],
)


def kernel(x, edge_index, W1, b1, Wmu, bmu, Wls, bls):
    src_p = edge_index[0].reshape(NW, PBLK, EBP)
    dst_p = edge_index[1].reshape(NW, PBLK, EBP)
    dst_d = edge_index[1].reshape(NW, NBLK, EB)
    degp = _sc_degree()(dst_d)
    xp = _t1(degp, x)
    prop = _sc_prop()
    acc1 = prop(xp, src_p, dst_p)
    hp = _t2(acc1, xp, degp, W1, b1.reshape(1, D))
    acc2 = prop(hp, src_p, dst_p)
    mu, logstd = _t3(acc2, hp, degp, Wmu, bmu.reshape(1, DO),
                     Wls, bls.reshape(1, DO))
    return (mu, logstd)
